# initial kernel scaffold (unmeasured)
import jax
import jax.numpy as jnp
from jax import lax
from jax.experimental import pallas as pl
from jax.experimental.pallas import tpu as pltpu

N_DEV = 4
N_HOP = N_DEV - 1


def kernel(x, Win0, Wout0, Win1, Wout1, Win2, Wout2):
    m_per, d = x.shape
    h_per = Win0.shape[1]
    M = N_DEV * m_per

    def body(x_ref, win0_ref, wout0_ref, win1_ref, wout1_ref, win2_ref,
             wout2_ref, out_ref,
             X_ref, psend_ref, pr_ref, acc_ref,
             xg_ssem, xg_rsem, p_ssem, p_rsem):
        my = lax.axis_index("i")
        left = (my - 1) % N_DEV
        right = (my + 1) % N_DEV

        barrier_sem = pltpu.get_barrier_semaphore()
        for nbr in (left, right):
            pl.semaphore_signal(
                barrier_sem, inc=1,
                device_id=(nbr,), device_id_type=pl.DeviceIdType.MESH,
            )
        pl.semaphore_wait(barrier_sem, 2)

        X_ref[pl.ds(my * m_per, m_per), :] = x_ref[...].astype(jnp.bfloat16)
        for h in range(N_HOP):
            o_send = (my - h) % N_DEV
            rdma = pltpu.make_async_remote_copy(
                src_ref=X_ref.at[pl.ds(o_send * m_per, m_per), :],
                dst_ref=X_ref.at[pl.ds(o_send * m_per, m_per), :],
                send_sem=xg_ssem.at[h],
                recv_sem=xg_rsem.at[h],
                device_id=(right,),
                device_id_type=pl.DeviceIdType.MESH,
            )
            rdma.start()
            rdma.wait()

        for l, (win_ref, wout_ref) in enumerate(
            ((win0_ref, wout0_ref), (win1_ref, wout1_ref), (win2_ref, wout2_ref))
        ):
            X = X_ref[...]
            hmat = jnp.dot(
                X, win_ref[...].astype(jnp.bfloat16),
                preferred_element_type=jnp.float32,
            )
            hmat = jnp.maximum(hmat, 0.0).astype(jnp.bfloat16)
            P = jnp.dot(
                hmat, wout_ref[...].astype(jnp.bfloat16),
                preferred_element_type=jnp.float32,
            )
            psend_ref[...] = P.astype(jnp.bfloat16)
            acc_ref[...] = psend_ref[...].astype(jnp.float32)

            src = psend_ref
            for h in range(N_HOP):
                slot = l * N_HOP + h
                rdma = pltpu.make_async_remote_copy(
                    src_ref=src,
                    dst_ref=pr_ref.at[slot],
                    send_sem=p_ssem.at[slot],
                    recv_sem=p_rsem.at[slot],
                    device_id=(right,),
                    device_id_type=pl.DeviceIdType.MESH,
                )
                rdma.start()
                rdma.wait()
                acc_ref[...] += pr_ref[slot].astype(jnp.float32)
                src = pr_ref.at[slot]

            if l < 2:
                X_ref[...] = acc_ref[...].astype(jnp.bfloat16)

        out_ref[...] = acc_ref[...]

    return pl.pallas_call(
        body,
        out_shape=jax.ShapeDtypeStruct((M, d), jnp.float32),
        in_specs=[pl.BlockSpec(memory_space=pltpu.VMEM)] * 7,
        out_specs=pl.BlockSpec(memory_space=pltpu.VMEM),
        scratch_shapes=[
            pltpu.VMEM((M, d), jnp.bfloat16),
            pltpu.VMEM((M, d), jnp.bfloat16),
            pltpu.VMEM((3 * N_HOP, M, d), jnp.bfloat16),
            pltpu.VMEM((M, d), jnp.float32),
            pltpu.SemaphoreType.DMA((N_HOP,)),
            pltpu.SemaphoreType.DMA((N_HOP,)),
            pltpu.SemaphoreType.DMA((3 * N_HOP,)),
            pltpu.SemaphoreType.DMA((3 * N_HOP,)),
        ],
        compiler_params=pltpu.CompilerParams(collective_id=0),
    )(x, Win0, Wout0, Win1, Wout1, Win2, Wout2)


# baseline (device time: 122713 ns/iter reference)
import jax
import jax.numpy as jnp
from jax import lax
from jax.experimental import pallas as pl
from jax.experimental.pallas import tpu as pltpu

N_DEV = 4
N_HOP = N_DEV - 1


def kernel(x, Win0, Wout0, Win1, Wout1, Win2, Wout2):
    m_per, d = x.shape
    h_per = Win0.shape[1]
    M = N_DEV * m_per

    def body(x_ref, win0_ref, wout0_ref, win1_ref, wout1_ref, win2_ref,
             wout2_ref, out_ref,
             X_ref, psend_ref, pr_ref, acc_ref,
             xg_ssem, xg_rsem, p_ssem, p_rsem):
        my = lax.axis_index("i")
        left = (my - 1) % N_DEV
        right = (my + 1) % N_DEV

        barrier_sem = pltpu.get_barrier_semaphore()
        for nbr in (left, right):
            pl.semaphore_signal(
                barrier_sem, inc=1,
                device_id=(nbr,), device_id_type=pl.DeviceIdType.MESH,
            )
        pl.semaphore_wait(barrier_sem, 2)

        X_ref[pl.ds(my * m_per, m_per), :] = x_ref[...]
        for h in range(N_HOP):
            o_send = (my - h) % N_DEV
            rdma = pltpu.make_async_remote_copy(
                src_ref=X_ref.at[pl.ds(o_send * m_per, m_per), :],
                dst_ref=X_ref.at[pl.ds(o_send * m_per, m_per), :],
                send_sem=xg_ssem.at[h],
                recv_sem=xg_rsem.at[h],
                device_id=(right,),
                device_id_type=pl.DeviceIdType.MESH,
            )
            rdma.start()
            rdma.wait()

        for l, (win_ref, wout_ref) in enumerate(
            ((win0_ref, wout0_ref), (win1_ref, wout1_ref), (win2_ref, wout2_ref))
        ):
            X = X_ref[...]
            hmat = jnp.dot(
                X, win_ref[...], preferred_element_type=jnp.float32,
            )
            hmat = jnp.maximum(hmat, 0.0).astype(jnp.bfloat16)
            P = jnp.dot(
                hmat, wout_ref[...], preferred_element_type=jnp.float32,
            )
            psend_ref[...] = P.astype(jnp.bfloat16)
            acc_ref[...] = psend_ref[...].astype(jnp.float32)

            src = psend_ref
            for h in range(N_HOP):
                slot = l * N_HOP + h
                rdma = pltpu.make_async_remote_copy(
                    src_ref=src,
                    dst_ref=pr_ref.at[slot],
                    send_sem=p_ssem.at[slot],
                    recv_sem=p_rsem.at[slot],
                    device_id=(right,),
                    device_id_type=pl.DeviceIdType.MESH,
                )
                rdma.start()
                rdma.wait()
                acc_ref[...] += pr_ref[slot].astype(jnp.float32)
                src = pr_ref.at[slot]

            if l < 2:
                X_ref[...] = acc_ref[...].astype(jnp.bfloat16)

        out_ref[...] = acc_ref[...]

    return pl.pallas_call(
        body,
        out_shape=jax.ShapeDtypeStruct((M, d), jnp.float32),
        in_specs=[pl.BlockSpec(memory_space=pltpu.VMEM)] * 7,
        out_specs=pl.BlockSpec(memory_space=pltpu.VMEM),
        scratch_shapes=[
            pltpu.VMEM((M, d), jnp.bfloat16),
            pltpu.VMEM((M, d), jnp.bfloat16),
            pltpu.VMEM((3 * N_HOP, M, d), jnp.bfloat16),
            pltpu.VMEM((M, d), jnp.float32),
            pltpu.SemaphoreType.DMA((N_HOP,)),
            pltpu.SemaphoreType.DMA((N_HOP,)),
            pltpu.SemaphoreType.DMA((3 * N_HOP,)),
            pltpu.SemaphoreType.DMA((3 * N_HOP,)),
        ],
        compiler_params=pltpu.CompilerParams(
            collective_id=0,
            vmem_limit_bytes=100 * 1024 * 1024,
        ),
    )(
        x.astype(jnp.bfloat16),
        Win0.astype(jnp.bfloat16), Wout0.astype(jnp.bfloat16),
        Win1.astype(jnp.bfloat16), Wout1.astype(jnp.bfloat16),
        Win2.astype(jnp.bfloat16), Wout2.astype(jnp.bfloat16),
    )


# device time: 64988 ns/iter; 1.8882x vs baseline; 1.8882x over previous
import jax
import jax.numpy as jnp
from jax import lax
from jax.experimental import pallas as pl
from jax.experimental.pallas import tpu as pltpu

N_DEV = 4
N_PEER = N_DEV - 1
N_LAYER = 3


def kernel(x, Win0, Wout0, Win1, Wout1, Win2, Wout2):
    m_per, d = x.shape
    M = N_DEV * m_per

    def body(x_ref, win0_ref, wout0_ref, win1_ref, wout1_ref, win2_ref,
             wout2_ref, out_ref,
             X_ref, psend_ref, rs_ref,
             ag_ssem, ag_rsem, rs_ssem, rs_rsem):
        my = lax.axis_index("i")

        barrier_sem = pltpu.get_barrier_semaphore()
        for j in range(1, N_DEV):
            pl.semaphore_signal(
                barrier_sem, inc=1,
                device_id=((my + j) % N_DEV,),
                device_id_type=pl.DeviceIdType.MESH,
            )
        pl.semaphore_wait(barrier_sem, N_PEER)

        def ag_descriptor(phase, j, src_rows):
            peer = (my + j) % N_DEV
            return pltpu.make_async_remote_copy(
                src_ref=X_ref.at[pl.ds(src_rows * m_per, m_per), :],
                dst_ref=X_ref.at[pl.ds(src_rows * m_per, m_per), :],
                send_sem=ag_ssem.at[phase, j - 1],
                recv_sem=ag_rsem.at[phase, j - 1],
                device_id=(peer,),
                device_id_type=pl.DeviceIdType.MESH,
            )

        def run_allgather(phase):
            sends = []
            for j in range(1, N_DEV):
                s = ag_descriptor(phase, j, my)
                s.start()
                sends.append(s)
            for j in range(1, N_DEV):
                recv = pltpu.make_async_remote_copy(
                    src_ref=X_ref.at[pl.ds(((my + j) % N_DEV) * m_per, m_per), :],
                    dst_ref=X_ref.at[pl.ds(((my + j) % N_DEV) * m_per, m_per), :],
                    send_sem=ag_ssem.at[phase, j - 1],
                    recv_sem=ag_rsem.at[phase, (N_DEV - j) - 1],
                    device_id=(0,),
                    device_id_type=pl.DeviceIdType.MESH,
                )
                recv.wait_recv()
            for s in sends:
                s.wait_send()

        X_ref[pl.ds(my * m_per, m_per), :] = x_ref[...].astype(jnp.bfloat16)
        run_allgather(0)

        for l, (win_ref, wout_ref) in enumerate(
            ((win0_ref, wout0_ref), (win1_ref, wout1_ref), (win2_ref, wout2_ref))
        ):
            X = X_ref[...]
            hmat = jnp.dot(
                X, win_ref[...].astype(jnp.bfloat16),
                preferred_element_type=jnp.float32,
            )
            hmat = jnp.maximum(hmat, 0.0).astype(jnp.bfloat16)
            P = jnp.dot(
                hmat, wout_ref[...].astype(jnp.bfloat16),
                preferred_element_type=jnp.float32,
            )
            psend_ref[...] = P.astype(jnp.bfloat16)

            rs_sends = []
            for j in range(1, N_DEV):
                peer = (my + j) % N_DEV
                s = pltpu.make_async_remote_copy(
                    src_ref=psend_ref.at[pl.ds(peer * m_per, m_per), :],
                    dst_ref=rs_ref.at[l, j - 1],
                    send_sem=rs_ssem.at[l, j - 1],
                    recv_sem=rs_rsem.at[l, j - 1],
                    device_id=(peer,),
                    device_id_type=pl.DeviceIdType.MESH,
                )
                s.start()
                rs_sends.append(s)

            acc = psend_ref[pl.ds(my * m_per, m_per), :].astype(jnp.float32)
            for j in range(1, N_DEV):
                recv = pltpu.make_async_remote_copy(
                    src_ref=psend_ref.at[pl.ds(my * m_per, m_per), :],
                    dst_ref=rs_ref.at[l, (N_DEV - j) - 1],
                    send_sem=rs_ssem.at[l, j - 1],
                    recv_sem=rs_rsem.at[l, (N_DEV - j) - 1],
                    device_id=(0,),
                    device_id_type=pl.DeviceIdType.MESH,
                )
                recv.wait_recv()
                acc = acc + rs_ref[l, (N_DEV - j) - 1].astype(jnp.float32)

            X_ref[pl.ds(my * m_per, m_per), :] = acc.astype(jnp.bfloat16)
            for s in rs_sends:
                s.wait_send()
            run_allgather(l + 1)

        out_ref[...] = X_ref[...].astype(jnp.float32)

    return pl.pallas_call(
        body,
        out_shape=jax.ShapeDtypeStruct((M, d), jnp.float32),
        in_specs=[pl.BlockSpec(memory_space=pltpu.VMEM)] * 7,
        out_specs=pl.BlockSpec(memory_space=pltpu.VMEM),
        scratch_shapes=[
            pltpu.VMEM((M, d), jnp.bfloat16),
            pltpu.VMEM((M, d), jnp.bfloat16),
            pltpu.VMEM((N_LAYER, N_PEER, m_per, d), jnp.bfloat16),
            pltpu.SemaphoreType.DMA((N_LAYER + 1, N_PEER)),
            pltpu.SemaphoreType.DMA((N_LAYER + 1, N_PEER)),
            pltpu.SemaphoreType.DMA((N_LAYER, N_PEER)),
            pltpu.SemaphoreType.DMA((N_LAYER, N_PEER)),
        ],
        compiler_params=pltpu.CompilerParams(
            collective_id=0,
            vmem_limit_bytes=100 * 1024 * 1024,
        ),
    )(x, Win0, Wout0, Win1, Wout1, Win2, Wout2)
